# Initial kernel scaffold; baseline (speedup 1.0000x reference)
import functools

import jax
import jax.numpy as jnp
from jax import lax
from jax.experimental import pallas as pl
from jax.experimental.pallas import tpu as pltpu
from jax.experimental.pallas import tpu_sc as plsc

N = 10000
E = 320000
D = 128

NC = 2
NS = 16
NW = NC * NS
EPW = E // NW
CHUNK = 80
NCHUNK = EPW // CHUNK
RPT = N // NS
ZROWS = 25
LANES = 16


def _sc_partials(row, col, val, embeds):
    mesh = plsc.VectorSubcoreMesh(core_axis_name="c", subcore_axis_name="s")

    @functools.partial(
        pl.kernel,
        mesh=mesh,
        out_type=jax.ShapeDtypeStruct((NC, N, D), jnp.float32),
        scratch_types=[
            pltpu.VMEM((CHUNK,), jnp.int32),
            pltpu.VMEM((CHUNK,), jnp.int32),
            pltpu.VMEM((CHUNK,), jnp.float32),
            pltpu.VMEM((CHUNK, D), jnp.float32),
            pltpu.VMEM((ZROWS, D), jnp.float32),
            pltpu.VMEM_SHARED((N, D), jnp.float32),
            pltpu.SemaphoreType.DMA,
        ],
    )
    def k(row_hbm, col_hbm, val_hbm, emb_hbm, out_hbm,
          col_v, row_v, val_v, rows_v, zbuf, acc, sem):
        c = lax.axis_index("c")
        s = lax.axis_index("s")
        wid = c * NS + s
        ebase = wid * EPW

        zero = jnp.zeros((LANES,), jnp.float32)
        for i in range(ZROWS):
            for j in range(D // LANES):
                zbuf[i, pl.ds(j * LANES, LANES)] = zero
        for t in range(RPT // ZROWS):
            pltpu.sync_copy(zbuf, acc.at[pl.ds(s * RPT + t * ZROWS, ZROWS)])
        plsc.subcore_barrier()

        def chunk_body(kk, carry):
            base = ebase + kk * CHUNK
            pltpu.sync_copy(col_hbm.at[pl.ds(base, CHUNK)], col_v)
            pltpu.sync_copy(row_hbm.at[pl.ds(base, CHUNK)], row_v)
            pltpu.sync_copy(val_hbm.at[pl.ds(base, CHUNK)], val_v)
            pltpu.async_copy(emb_hbm.at[col_v], rows_v, sem).wait()
            for e in range(CHUNK):
                vsplat = plsc.load_gather(
                    val_v, [jnp.full((LANES,), e, jnp.int32)])
                for j in range(D // LANES):
                    sl = pl.ds(j * LANES, LANES)
                    rows_v[e, sl] = rows_v[e, sl] * vsplat
            pltpu.sync_copy(rows_v, acc.at[row_v], add=True)
            return carry

        lax.fori_loop(0, NCHUNK, chunk_body, 0)
        plsc.subcore_barrier()
        pltpu.sync_copy(acc.at[pl.ds(s * RPT, RPT)],
                        out_hbm.at[c, pl.ds(s * RPT, RPT)])

    return k(row, col, val, embeds)


def _combine(partials):
    def body(p_ref, o_ref):
        o_ref[...] = p_ref[0] + p_ref[1]

    rblk = 1000
    return pl.pallas_call(
        body,
        out_shape=jax.ShapeDtypeStruct((N, D), jnp.float32),
        grid=(N // rblk,),
        in_specs=[pl.BlockSpec((NC, rblk, D), lambda i: (0, i, 0))],
        out_specs=pl.BlockSpec((rblk, D), lambda i: (i, 0)),
    )(partials)


def kernel(adj_indices, adj_values, embeds):
    row = adj_indices[0]
    col = adj_indices[1]
    partials = _sc_partials(row, col, adj_values, embeds)
    return _combine(partials)


# speedup vs baseline: 4.5624x; 4.5624x over previous
import functools

import jax
import jax.numpy as jnp
from jax import lax
from jax.experimental import pallas as pl
from jax.experimental.pallas import tpu as pltpu
from jax.experimental.pallas import tpu_sc as plsc

N = 10000
E = 320000
D = 128

NC = 2
NS = 16
NW = NC * NS
EPW = E // NW
CHUNK = 80
NCHUNK = EPW // CHUNK
NP = 10240
RPT = NP // NS
ZROWS = 40
LANES = 16


def _lane_splat(vec, i):
    idx = jnp.full((LANES, 1), i, jnp.int32)
    dnums = lax.GatherDimensionNumbers(
        offset_dims=(), collapsed_slice_dims=(0,), start_index_map=(0,))
    return lax.gather(vec, idx, dnums, (1,),
                      mode=lax.GatherScatterMode.PROMISE_IN_BOUNDS)


def _sc_partials(row, col, val, embeds):
    mesh = plsc.VectorSubcoreMesh(core_axis_name="c", subcore_axis_name="s")

    @functools.partial(
        pl.kernel,
        mesh=mesh,
        out_type=jax.ShapeDtypeStruct((NC, NP, D), jnp.float32),
        scratch_types=[
            pltpu.VMEM((CHUNK,), jnp.int32),
            pltpu.VMEM((CHUNK,), jnp.int32),
            pltpu.VMEM((CHUNK,), jnp.float32),
            pltpu.VMEM((CHUNK, D), jnp.float32),
            pltpu.VMEM((ZROWS, D), jnp.float32),
            pltpu.VMEM_SHARED((NP, D), jnp.float32),
            pltpu.SemaphoreType.DMA,
        ],
    )
    def k(row_hbm, col_hbm, val_hbm, emb_hbm, out_hbm,
          col_v, row_v, val_v, rows_v, zbuf, acc, sem):
        c = lax.axis_index("c")
        s = lax.axis_index("s")
        wid = c * NS + s
        ebase = wid * EPW

        zero = jnp.zeros((LANES,), jnp.float32)
        for i in range(ZROWS):
            for j in range(D // LANES):
                zbuf[i, pl.ds(j * LANES, LANES)] = zero
        for t in range(RPT // ZROWS):
            pltpu.sync_copy(zbuf, acc.at[pl.ds(s * RPT + t * ZROWS, ZROWS)])
        plsc.subcore_barrier()

        def chunk_body(kk, carry):
            base = ebase + kk * CHUNK
            pltpu.sync_copy(col_hbm.at[pl.ds(base, CHUNK)], col_v)
            pltpu.sync_copy(row_hbm.at[pl.ds(base, CHUNK)], row_v)
            pltpu.sync_copy(val_hbm.at[pl.ds(base, CHUNK)], val_v)
            pltpu.async_copy(emb_hbm.at[col_v], rows_v, sem).wait()
            for g in range(CHUNK // LANES):
                val16 = val_v[pl.ds(g * LANES, LANES)]
                for i in range(LANES):
                    e = g * LANES + i
                    vsplat = _lane_splat(val16, i)
                    for j in range(D // LANES):
                        sl = pl.ds(j * LANES, LANES)
                        rows_v[e, sl] = rows_v[e, sl] * vsplat
            pltpu.sync_copy(rows_v, acc.at[row_v], add=True)
            return carry

        lax.fori_loop(0, NCHUNK, chunk_body, 0)
        plsc.subcore_barrier()
        pltpu.sync_copy(acc.at[pl.ds(s * RPT, RPT)],
                        out_hbm.at[c, pl.ds(s * RPT, RPT)])

    return k(row, col, val, embeds)


def _combine(partials):
    def body(p_ref, o_ref):
        o_ref[...] = p_ref[0] + p_ref[1]

    rblk = 1000
    return pl.pallas_call(
        body,
        out_shape=jax.ShapeDtypeStruct((N, D), jnp.float32),
        grid=(N // rblk,),
        in_specs=[pl.BlockSpec((NC, rblk, D), lambda i: (0, i, 0))],
        out_specs=pl.BlockSpec((rblk, D), lambda i: (i, 0)),
    )(partials)


def kernel(adj_indices, adj_values, embeds):
    row = adj_indices[0]
    col = adj_indices[1]
    partials = _sc_partials(row, col, adj_values, embeds)
    return _combine(partials)


# speedup vs baseline: 12.1005x; 2.6522x over previous
import functools

import jax
import jax.numpy as jnp
from jax import lax
from jax.experimental import pallas as pl
from jax.experimental.pallas import tpu as pltpu
from jax.experimental.pallas import tpu_sc as plsc

N = 10000
E = 320000
D = 128

NC = 2
NS = 16
NW = NC * NS
EPW = E // NW
CHUNK = 80
NCHUNK = EPW // CHUNK
NBUF = 3
NITER = 41
GROUPS = CHUNK // 16
NP = 10240
RPT = NP // NS
LANES = 16


def _lane_splat(vec, i):
    idx = jnp.full((LANES, 1), i, jnp.int32)
    dnums = lax.GatherDimensionNumbers(
        offset_dims=(), collapsed_slice_dims=(0,), start_index_map=(0,))
    return lax.gather(vec, idx, dnums, (1,),
                      mode=lax.GatherScatterMode.PROMISE_IN_BOUNDS)


def _sc_partials(row3, col3, val3, embeds):
    mesh = plsc.VectorSubcoreMesh(core_axis_name="c", subcore_axis_name="s")

    @functools.partial(
        pl.kernel,
        mesh=mesh,
        out_type=jax.ShapeDtypeStruct((NC, NP, D), jnp.float32),
        scratch_types=(
            [pltpu.VMEM_SHARED((NP, D), jnp.float32)]
            + [pltpu.VMEM((NCHUNK, CHUNK), jnp.int32)]
            + [pltpu.VMEM((CHUNK, D), jnp.float32) for _ in range(NBUF)]
            + [pltpu.VMEM((CHUNK,), jnp.int32) for _ in range(NBUF)]
            + [pltpu.VMEM((CHUNK,), jnp.float32) for _ in range(NBUF)]
            + [pltpu.SemaphoreType.DMA for _ in range(3 * NBUF)]
        ),
    )
    def k(row_hbm, col_hbm, val_hbm, emb_hbm, out_hbm, acc, row_all,
          *bufs_sems):
        bufs = bufs_sems[:NBUF]
        mbc = bufs_sems[NBUF:2 * NBUF]
        mbv = bufs_sems[2 * NBUF:3 * NBUF]
        gsem = bufs_sems[3 * NBUF:4 * NBUF]
        ssem = bufs_sems[4 * NBUF:5 * NBUF]
        msem = bufs_sems[5 * NBUF:]
        c = lax.axis_index("c")
        s = lax.axis_index("s")
        wid = c * NS + s

        pltpu.sync_copy(row_hbm.at[wid], row_all)

        zero = jnp.zeros((LANES,), jnp.float32)
        for i in range(CHUNK):
            for j in range(D // LANES):
                bufs[0][i, pl.ds(j * LANES, LANES)] = zero
        for t in range(RPT // CHUNK):
            pltpu.sync_copy(bufs[0],
                            acc.at[pl.ds(s * RPT + t * CHUNK, CHUNK)])
        plsc.subcore_barrier()

        def start_meta(kk, b):
            pltpu.async_copy(col_hbm.at[wid, kk], mbc[b], msem[b])
            pltpu.async_copy(val_hbm.at[wid, kk], mbv[b], msem[b])

        def wait_meta(kk, b):
            pltpu.make_async_copy(col_hbm.at[wid, kk], mbc[b],
                                  msem[b]).wait()
            pltpu.make_async_copy(val_hbm.at[wid, kk], mbv[b],
                                  msem[b]).wait()

        def start_gather(kk, b):
            pltpu.async_copy(emb_hbm.at[mbc[b]], bufs[b], gsem[b])

        def wait_gather(kk, b):
            pltpu.make_async_copy(
                emb_hbm.at[mbc[b]], bufs[b], gsem[b]).wait()

        def start_scat(kk, b):
            pltpu.async_copy(bufs[b], acc.at[row_all.at[kk]], ssem[b],
                             add=True)

        def wait_scat(kk, b):
            pltpu.make_async_copy(
                bufs[b], acc.at[row_all.at[kk]], ssem[b]).wait()

        def scale(kk, b):
            def gbody(g, carry):
                val16 = mbv[b][pl.ds(g * LANES, LANES)]
                for i in range(LANES):
                    e = g * LANES + i
                    vsplat = _lane_splat(val16, i)
                    for j in range(D // LANES):
                        sl = pl.ds(j * LANES, LANES)
                        bufs[b][e, sl] = bufs[b][e, sl] * vsplat
                return carry
            lax.fori_loop(0, GROUPS, gbody, 0)

        for j in range(NBUF):
            start_meta(j, j)
        for j in range(NBUF):
            wait_meta(j, j)
            start_gather(j, j)

        def chunk_body(m, carry):
            for j in range(NBUF):
                kk = m * NBUF + j
                wait_gather(kk, j)
                scale(kk, j)

                @pl.when(kk + NBUF <= NCHUNK - 1)
                def _():
                    start_meta(kk + NBUF, j)

                start_scat(kk, j)

                pj = (j - 1) % NBUF

                @pl.when((m > 0) if j == 0 else (kk <= NCHUNK - 3))
                def _():
                    wait_scat(kk - 1, pj)
                    wait_meta(kk + NBUF - 1, pj)
                    start_gather(kk + NBUF - 1, pj)
            return carry

        lax.fori_loop(0, NITER, chunk_body, 0)

        wait_scat(NCHUNK - 3, 2)
        wait_gather(NCHUNK - 2, 0)
        scale(NCHUNK - 2, 0)
        start_scat(NCHUNK - 2, 0)
        wait_gather(NCHUNK - 1, 1)
        scale(NCHUNK - 1, 1)
        start_scat(NCHUNK - 1, 1)
        wait_scat(NCHUNK - 2, 0)
        wait_scat(NCHUNK - 1, 1)

        plsc.subcore_barrier()
        pltpu.sync_copy(acc.at[pl.ds(s * RPT, RPT)],
                        out_hbm.at[c, pl.ds(s * RPT, RPT)])

    return k(row3, col3, val3, embeds)


def _combine(partials):
    def body(p_ref, o_ref):
        o_ref[...] = p_ref[0] + p_ref[1]

    rblk = 1000
    return pl.pallas_call(
        body,
        out_shape=jax.ShapeDtypeStruct((N, D), jnp.float32),
        grid=(N // rblk,),
        in_specs=[pl.BlockSpec((NC, rblk, D), lambda i: (0, i, 0))],
        out_specs=pl.BlockSpec((rblk, D), lambda i: (i, 0)),
    )(partials)


def kernel(adj_indices, adj_values, embeds):
    row3 = adj_indices[0].reshape(NW, NCHUNK, CHUNK)
    col3 = adj_indices[1].reshape(NW, NCHUNK, CHUNK)
    val3 = adj_values.reshape(NW, NCHUNK, CHUNK)
    partials = _sc_partials(row3, col3, val3, embeds)
    return _combine(partials)
